# Initial kernel scaffold; baseline (speedup 1.0000x reference)
import jax
import jax.numpy as jnp
from jax.experimental import pallas as pl


def kernel(x, edge_indices, W, a):
    raise NotImplementedError("write your pallas kernel here")


# speedup vs baseline: 188.6161x; 188.6161x over previous
import functools

import jax
import jax.numpy as jnp
from jax import lax
from jax.experimental import pallas as pl
from jax.experimental.pallas import tpu as pltpu
from jax.experimental.pallas import tpu_sc as plsc

N = 10000
E = 160000
IN_F = 128
HEADS = 8
OUT_F = 16

NC = 2
NS = 16
NW = NC * NS
CHUNK = 128
CHUNKS_PER_TILE = 40
E_PAD = NW * CHUNKS_PER_TILE * CHUNK
N_PAD = 10112
ROWS_PER_TILE = N_PAD // NS

MM_BLOCK = 1000


def _mm_body(x_ref, w_ref, y_ref):
    w = w_ref[...]
    w2 = w[:, 0:OUT_F]
    for h in range(1, HEADS):
        w2 = w2 + w[:, h * OUT_F:(h + 1) * OUT_F]
    y_ref[...] = jnp.dot(x_ref[...], w2,
                         preferred_element_type=jnp.float32) * (1.0 / HEADS)


def _matmul(x2, W):
    return pl.pallas_call(
        _mm_body,
        grid=(N // MM_BLOCK,),
        in_specs=[
            pl.BlockSpec((MM_BLOCK, IN_F), lambda i: (i, 0)),
            pl.BlockSpec((IN_F, IN_F), lambda i: (0, 0)),
        ],
        out_specs=pl.BlockSpec((MM_BLOCK, OUT_F), lambda i: (i, 0)),
        out_shape=jax.ShapeDtypeStruct((N, OUT_F), jnp.float32),
    )(x2, W)


def _sc_body(y_hbm, src_hbm, dst_hbm, zeros_hbm, part_hbm,
             src_v, dst_v, row_buf, acc, sem):
    c = lax.axis_index("c")
    s = lax.axis_index("s")
    wid = c * NS + s
    stripe = pl.ds(s * ROWS_PER_TILE, ROWS_PER_TILE)
    pltpu.sync_copy(zeros_hbm.at[stripe], acc.at[stripe])
    pltpu.sync_copy(src_hbm.at[wid], src_v)
    pltpu.sync_copy(dst_hbm.at[wid], dst_v)
    plsc.subcore_barrier()

    def body(j, carry):
        pltpu.async_copy(y_hbm.at[dst_v.at[j]], row_buf, sem).wait()
        pltpu.sync_copy(row_buf, acc.at[src_v.at[j]], add=True)
        return carry

    lax.fori_loop(0, CHUNKS_PER_TILE, body, 0)
    plsc.subcore_barrier()
    pltpu.sync_copy(acc.at[stripe], part_hbm.at[c, stripe])


def _scatter(y, src_p, dst_p, zeros):
    mesh = plsc.VectorSubcoreMesh(core_axis_name="c", subcore_axis_name="s")
    f = pl.kernel(
        _sc_body,
        out_type=jax.ShapeDtypeStruct((NC, N_PAD, OUT_F), jnp.float32),
        mesh=mesh,
        scratch_types=[
            pltpu.VMEM((CHUNKS_PER_TILE, CHUNK), jnp.int32),
            pltpu.VMEM((CHUNKS_PER_TILE, CHUNK), jnp.int32),
            pltpu.VMEM((CHUNK, OUT_F), jnp.float32),
            pltpu.VMEM_SHARED((N_PAD, OUT_F), jnp.float32),
            pltpu.SemaphoreType.DMA,
        ],
        compiler_params=pltpu.CompilerParams(use_tc_tiling_on_sc=False),
    )
    return f(y, src_p, dst_p, zeros)


def _combine_body(p0_ref, p1_ref, out_ref):
    out_ref[...] = p0_ref[0] + p1_ref[0]


def _combine(part):
    return pl.pallas_call(
        _combine_body,
        grid=(N // MM_BLOCK,),
        in_specs=[
            pl.BlockSpec((1, MM_BLOCK, OUT_F), lambda i: (0, i, 0)),
            pl.BlockSpec((1, MM_BLOCK, OUT_F), lambda i: (1, i, 0)),
        ],
        out_specs=pl.BlockSpec((MM_BLOCK, OUT_F), lambda i: (i, 0)),
        out_shape=jax.ShapeDtypeStruct((N, OUT_F), jnp.float32),
    )(part, part)


def kernel(x, edge_indices, W, a):
    del a
    x2 = x[0]
    y = _matmul(x2, W)

    pad = E_PAD - E
    src_p = jnp.concatenate(
        [edge_indices[0], jnp.full((pad,), N, jnp.int32)]
    ).reshape(NW, CHUNKS_PER_TILE, CHUNK)
    dst_p = jnp.concatenate(
        [edge_indices[1], jnp.zeros((pad,), jnp.int32)]
    ).reshape(NW, CHUNKS_PER_TILE, CHUNK)
    zeros = jnp.zeros((N_PAD, OUT_F), jnp.float32)

    part = _scatter(y, src_p, dst_p, zeros)
    out = _combine(part)
    return out[None]


# speedup vs baseline: 229.2157x; 1.2153x over previous
import functools

import jax
import jax.numpy as jnp
from jax import lax
from jax.experimental import pallas as pl
from jax.experimental.pallas import tpu as pltpu
from jax.experimental.pallas import tpu_sc as plsc

N = 10000
E = 160000
IN_F = 128
HEADS = 8
OUT_F = 16

NC = 2
NS = 16
NW = NC * NS
CHUNK = 128
CHUNKS_PER_TILE = 40
E_PAD = NW * CHUNKS_PER_TILE * CHUNK
N_PAD = 10112
ROWS_PER_TILE = N_PAD // NS

MM_BLOCK = 1000


def _mm_body(x_ref, w_ref, y_ref):
    w = w_ref[...]
    w2 = w[:, 0:OUT_F]
    for h in range(1, HEADS):
        w2 = w2 + w[:, h * OUT_F:(h + 1) * OUT_F]
    y_ref[...] = jnp.dot(x_ref[...], w2,
                         preferred_element_type=jnp.float32) * (1.0 / HEADS)


def _matmul(x2, W):
    return pl.pallas_call(
        _mm_body,
        grid=(N // MM_BLOCK,),
        in_specs=[
            pl.BlockSpec((MM_BLOCK, IN_F), lambda i: (i, 0)),
            pl.BlockSpec((IN_F, IN_F), lambda i: (0, 0)),
        ],
        out_specs=pl.BlockSpec((MM_BLOCK, OUT_F), lambda i: (i, 0)),
        out_shape=jax.ShapeDtypeStruct((N, OUT_F), jnp.float32),
    )(x2, W)


NBUF = 4


def _sc_body(y_hbm, src_hbm, dst_hbm, zeros_hbm, part_hbm,
             src_v, dst_v, row_buf, acc, sem):
    c = lax.axis_index("c")
    s = lax.axis_index("s")
    wid = c * NS + s
    stripe = pl.ds(s * ROWS_PER_TILE, ROWS_PER_TILE)
    pltpu.sync_copy(zeros_hbm.at[stripe], acc.at[stripe])
    pltpu.sync_copy(src_hbm.at[wid], src_v)
    pltpu.sync_copy(dst_hbm.at[wid], dst_v)
    plsc.subcore_barrier()

    for b in range(NBUF):
        pltpu.async_copy(y_hbm.at[dst_v.at[b]], row_buf.at[b], sem.at[b])

    def outer(j0, carry):
        for b in range(NBUF):
            j = j0 * NBUF + b
            pltpu.make_async_copy(
                y_hbm.at[dst_v.at[j]], row_buf.at[b], sem.at[b]).wait()
            pltpu.sync_copy(row_buf.at[b], acc.at[src_v.at[j]], add=True)
            jn = j + NBUF

            @pl.when(jn < CHUNKS_PER_TILE)
            def _():
                pltpu.async_copy(
                    y_hbm.at[dst_v.at[jn]], row_buf.at[b], sem.at[b])
        return carry

    lax.fori_loop(0, CHUNKS_PER_TILE // NBUF, outer, 0)
    plsc.subcore_barrier()
    pltpu.sync_copy(acc.at[stripe], part_hbm.at[c, stripe])


def _scatter(y, src_p, dst_p, zeros):
    mesh = plsc.VectorSubcoreMesh(core_axis_name="c", subcore_axis_name="s")
    f = pl.kernel(
        _sc_body,
        out_type=jax.ShapeDtypeStruct((NC, N_PAD, OUT_F), jnp.float32),
        mesh=mesh,
        scratch_types=[
            pltpu.VMEM((CHUNKS_PER_TILE, CHUNK), jnp.int32),
            pltpu.VMEM((CHUNKS_PER_TILE, CHUNK), jnp.int32),
            pltpu.VMEM((NBUF, CHUNK, OUT_F), jnp.float32),
            pltpu.VMEM_SHARED((N_PAD, OUT_F), jnp.float32),
            pltpu.SemaphoreType.DMA((NBUF,)),
        ],
        compiler_params=pltpu.CompilerParams(use_tc_tiling_on_sc=False),
    )
    return f(y, src_p, dst_p, zeros)


def _combine_body(p0_ref, p1_ref, out_ref):
    out_ref[...] = p0_ref[0] + p1_ref[0]


def _combine(part):
    return pl.pallas_call(
        _combine_body,
        grid=(N // MM_BLOCK,),
        in_specs=[
            pl.BlockSpec((1, MM_BLOCK, OUT_F), lambda i: (0, i, 0)),
            pl.BlockSpec((1, MM_BLOCK, OUT_F), lambda i: (1, i, 0)),
        ],
        out_specs=pl.BlockSpec((MM_BLOCK, OUT_F), lambda i: (i, 0)),
        out_shape=jax.ShapeDtypeStruct((N, OUT_F), jnp.float32),
    )(part, part)


def kernel(x, edge_indices, W, a):
    del a
    x2 = x[0]
    y = _matmul(x2, W)

    pad = E_PAD - E
    src_p = jnp.concatenate(
        [edge_indices[0], jnp.full((pad,), N, jnp.int32)]
    ).reshape(NW, CHUNKS_PER_TILE, CHUNK)
    dst_p = jnp.concatenate(
        [edge_indices[1], jnp.zeros((pad,), jnp.int32)]
    ).reshape(NW, CHUNKS_PER_TILE, CHUNK)
    zeros = jnp.zeros((N_PAD, OUT_F), jnp.float32)

    part = _scatter(y, src_p, dst_p, zeros)
    out = _combine(part)
    return out[None]


# speedup vs baseline: 255.4250x; 1.1143x over previous
import functools

import jax
import jax.numpy as jnp
from jax import lax
from jax.experimental import pallas as pl
from jax.experimental.pallas import tpu as pltpu
from jax.experimental.pallas import tpu_sc as plsc

N = 10000
E = 160000
IN_F = 128
HEADS = 8
OUT_F = 16

NS = 16
CHUNK = 128
CHUNKS_PER_TILE = 80
E_PAD = NS * CHUNKS_PER_TILE * CHUNK
N_PAD = 10112
ROWS_PER_TILE = N_PAD // NS
LAST_ROWS = N - (NS - 1) * ROWS_PER_TILE

MM_BLOCK = 1000
NBUF = 4


def _mm_body(x_ref, w_ref, y_ref):
    w = w_ref[...]
    w2 = w[:, 0:OUT_F]
    for h in range(1, HEADS):
        w2 = w2 + w[:, h * OUT_F:(h + 1) * OUT_F]
    y_ref[...] = jnp.dot(x_ref[...], w2,
                         preferred_element_type=jnp.float32) * (1.0 / HEADS)


def _matmul(x2, W):
    return pl.pallas_call(
        _mm_body,
        grid=(N // MM_BLOCK,),
        in_specs=[
            pl.BlockSpec((MM_BLOCK, IN_F), lambda i: (i, 0)),
            pl.BlockSpec((IN_F, IN_F), lambda i: (0, 0)),
        ],
        out_specs=pl.BlockSpec((MM_BLOCK, OUT_F), lambda i: (i, 0)),
        out_shape=jax.ShapeDtypeStruct((N, OUT_F), jnp.float32),
    )(x2, W)


def _sc_body(y_hbm, src_hbm, dst_hbm, zeros_hbm, out_hbm,
             src_v, dst_v, row_buf, acc, sem):
    s = lax.axis_index("s")
    stripe = pl.ds(s * ROWS_PER_TILE, ROWS_PER_TILE)
    pltpu.sync_copy(zeros_hbm.at[stripe], acc.at[stripe])
    pltpu.sync_copy(src_hbm.at[s], src_v)
    pltpu.sync_copy(dst_hbm.at[s], dst_v)
    plsc.subcore_barrier()

    for b in range(NBUF):
        pltpu.async_copy(y_hbm.at[dst_v.at[b]], row_buf.at[b], sem.at[b])

    def outer(j0, carry):
        for b in range(NBUF):
            j = j0 * NBUF + b
            pltpu.make_async_copy(
                y_hbm.at[dst_v.at[j]], row_buf.at[b], sem.at[b]).wait()
            pltpu.sync_copy(row_buf.at[b], acc.at[src_v.at[j]], add=True)
            jn = j + NBUF

            @pl.when(jn < CHUNKS_PER_TILE)
            def _():
                pltpu.async_copy(
                    y_hbm.at[dst_v.at[jn]], row_buf.at[b], sem.at[b])
        return carry

    lax.fori_loop(0, CHUNKS_PER_TILE // NBUF, outer, 0)
    plsc.subcore_barrier()

    @pl.when(s < NS - 1)
    def _():
        pltpu.sync_copy(acc.at[stripe], out_hbm.at[stripe])

    @pl.when(s == NS - 1)
    def _():
        last = pl.ds((NS - 1) * ROWS_PER_TILE, LAST_ROWS)
        pltpu.sync_copy(acc.at[last], out_hbm.at[last])


def _scatter(y, src_p, dst_p, zeros):
    mesh = plsc.VectorSubcoreMesh(
        core_axis_name="c", subcore_axis_name="s", num_cores=1)
    f = pl.kernel(
        _sc_body,
        out_type=jax.ShapeDtypeStruct((N, OUT_F), jnp.float32),
        mesh=mesh,
        scratch_types=[
            pltpu.VMEM((CHUNKS_PER_TILE, CHUNK), jnp.int32),
            pltpu.VMEM((CHUNKS_PER_TILE, CHUNK), jnp.int32),
            pltpu.VMEM((NBUF, CHUNK, OUT_F), jnp.float32),
            pltpu.VMEM_SHARED((N_PAD, OUT_F), jnp.float32),
            pltpu.SemaphoreType.DMA((NBUF,)),
        ],
        compiler_params=pltpu.CompilerParams(use_tc_tiling_on_sc=False),
    )
    return f(y, src_p, dst_p, zeros)


def kernel(x, edge_indices, W, a):
    del a
    x2 = x[0]
    y = _matmul(x2, W)

    pad = E_PAD - E
    src_p = jnp.concatenate(
        [edge_indices[0], jnp.full((pad,), N, jnp.int32)]
    ).reshape(NS, CHUNKS_PER_TILE, CHUNK)
    dst_p = jnp.concatenate(
        [edge_indices[1], jnp.zeros((pad,), jnp.int32)]
    ).reshape(NS, CHUNKS_PER_TILE, CHUNK)
    zeros = jnp.zeros((N_PAD, OUT_F), jnp.float32)

    out = _scatter(y, src_p, dst_p, zeros)
    return out[None]


# speedup vs baseline: 267.5910x; 1.0476x over previous
import functools

import jax
import jax.numpy as jnp
from jax import lax
from jax.experimental import pallas as pl
from jax.experimental.pallas import tpu as pltpu
from jax.experimental.pallas import tpu_sc as plsc

N = 10000
E = 160000
IN_F = 128
HEADS = 8
OUT_F = 16

NS = 16
CHUNK = 128
CHUNKS_PER_TILE = 80
E_PAD = NS * CHUNKS_PER_TILE * CHUNK
N_PAD = 10112
ROWS_PER_TILE = N_PAD // NS
LAST_ROWS = N - (NS - 1) * ROWS_PER_TILE

MM_BLOCK = 1000
NBUF = 8


def _mm_body(x_ref, w_ref, y_ref):
    w = w_ref[...]
    w2 = w[:, 0:OUT_F]
    for h in range(1, HEADS):
        w2 = w2 + w[:, h * OUT_F:(h + 1) * OUT_F]
    y_ref[...] = jnp.dot(x_ref[...], w2,
                         preferred_element_type=jnp.float32) * (1.0 / HEADS)


def _matmul(x2, W):
    return pl.pallas_call(
        _mm_body,
        grid=(N // MM_BLOCK,),
        in_specs=[
            pl.BlockSpec((MM_BLOCK, IN_F), lambda i: (i, 0)),
            pl.BlockSpec((IN_F, IN_F), lambda i: (0, 0)),
        ],
        out_specs=pl.BlockSpec((MM_BLOCK, OUT_F), lambda i: (i, 0)),
        out_shape=jax.ShapeDtypeStruct((N, OUT_F), jnp.float32),
    )(x2, W)


def _sc_body(y_hbm, src_hbm, dst_hbm, zeros_hbm, out_hbm,
             src_v, dst_v, row_buf, acc, sem):
    s = lax.axis_index("s")
    stripe = pl.ds(s * ROWS_PER_TILE, ROWS_PER_TILE)
    pltpu.sync_copy(zeros_hbm.at[stripe], acc.at[stripe])
    pltpu.sync_copy(src_hbm.at[s], src_v)
    pltpu.sync_copy(dst_hbm.at[s], dst_v)
    plsc.subcore_barrier()

    for b in range(NBUF):
        pltpu.async_copy(y_hbm.at[dst_v.at[b]], row_buf.at[b], sem.at[b])

    def outer(j0, carry):
        for b in range(NBUF):
            j = j0 * NBUF + b
            pltpu.make_async_copy(
                y_hbm.at[dst_v.at[j]], row_buf.at[b], sem.at[b]).wait()
            pltpu.sync_copy(row_buf.at[b], acc.at[src_v.at[j]], add=True)
            jn = j + NBUF

            @pl.when(jn < CHUNKS_PER_TILE)
            def _():
                pltpu.async_copy(
                    y_hbm.at[dst_v.at[jn]], row_buf.at[b], sem.at[b])
        return carry

    lax.fori_loop(0, CHUNKS_PER_TILE // NBUF, outer, 0)
    plsc.subcore_barrier()

    @pl.when(s < NS - 1)
    def _():
        pltpu.sync_copy(acc.at[stripe], out_hbm.at[stripe])

    @pl.when(s == NS - 1)
    def _():
        last = pl.ds((NS - 1) * ROWS_PER_TILE, LAST_ROWS)
        pltpu.sync_copy(acc.at[last], out_hbm.at[last])


def _scatter(y, src_p, dst_p, zeros):
    mesh = plsc.VectorSubcoreMesh(
        core_axis_name="c", subcore_axis_name="s", num_cores=1)
    f = pl.kernel(
        _sc_body,
        out_type=jax.ShapeDtypeStruct((N, OUT_F), jnp.float32),
        mesh=mesh,
        scratch_types=[
            pltpu.VMEM((CHUNKS_PER_TILE, CHUNK), jnp.int32),
            pltpu.VMEM((CHUNKS_PER_TILE, CHUNK), jnp.int32),
            pltpu.VMEM((NBUF, CHUNK, OUT_F), jnp.float32),
            pltpu.VMEM_SHARED((N_PAD, OUT_F), jnp.float32),
            pltpu.SemaphoreType.DMA((NBUF,)),
        ],
        compiler_params=pltpu.CompilerParams(use_tc_tiling_on_sc=False),
    )
    return f(y, src_p, dst_p, zeros)


def kernel(x, edge_indices, W, a):
    del a
    x2 = x[0]
    y = _matmul(x2, W)

    pad = E_PAD - E
    src_p = jnp.concatenate(
        [edge_indices[0], jnp.full((pad,), N, jnp.int32)]
    ).reshape(NS, CHUNKS_PER_TILE, CHUNK)
    dst_p = jnp.concatenate(
        [edge_indices[1], jnp.zeros((pad,), jnp.int32)]
    ).reshape(NS, CHUNKS_PER_TILE, CHUNK)
    zeros = jnp.zeros((N_PAD, OUT_F), jnp.float32)

    out = _scatter(y, src_p, dst_p, zeros)
    return out[None]


# speedup vs baseline: 309.6452x; 1.1572x over previous
import functools

import jax
import jax.numpy as jnp
from jax import lax
from jax.experimental import pallas as pl
from jax.experimental.pallas import tpu as pltpu
from jax.experimental.pallas import tpu_sc as plsc

N = 10000
E = 160000
IN_F = 128
HEADS = 8
OUT_F = 16

NS = 16
CHUNK = 128
CHUNKS_PER_TILE = 80
E_PAD = NS * CHUNKS_PER_TILE * CHUNK
N_PAD = 10112
ROWS_PER_TILE = N_PAD // NS
LAST_ROWS = N - (NS - 1) * ROWS_PER_TILE

MM_BLOCK = 1000
NBUF = 8


def _mm_body(x_ref, w_ref, y_ref):
    w = w_ref[...]
    w2 = w[:, 0:OUT_F]
    for h in range(1, HEADS):
        w2 = w2 + w[:, h * OUT_F:(h + 1) * OUT_F]
    y_ref[...] = jnp.dot(x_ref[...], w2,
                         preferred_element_type=jnp.float32) * (1.0 / HEADS)


def _matmul(x2, W):
    return pl.pallas_call(
        _mm_body,
        grid=(N // MM_BLOCK,),
        in_specs=[
            pl.BlockSpec((MM_BLOCK, IN_F), lambda i: (i, 0)),
            pl.BlockSpec((IN_F, IN_F), lambda i: (0, 0)),
        ],
        out_specs=pl.BlockSpec((MM_BLOCK, OUT_F), lambda i: (i, 0)),
        out_shape=jax.ShapeDtypeStruct((N, OUT_F), jnp.float32),
    )(x2, W)


def _sc_body(y_hbm, src_hbm, dst_hbm, zeros_hbm, out_hbm,
             src_v, dst_v, row_buf, acc, y_sh, sem):
    s = lax.axis_index("s")
    stripe = pl.ds(s * ROWS_PER_TILE, ROWS_PER_TILE)
    pltpu.sync_copy(zeros_hbm.at[stripe], acc.at[stripe])

    @pl.when(s < NS - 1)
    def _():
        pltpu.sync_copy(y_hbm.at[stripe], y_sh.at[stripe])

    @pl.when(s == NS - 1)
    def _():
        last = pl.ds((NS - 1) * ROWS_PER_TILE, LAST_ROWS)
        pltpu.sync_copy(y_hbm.at[last], y_sh.at[last])

    pltpu.sync_copy(src_hbm.at[s], src_v)
    pltpu.sync_copy(dst_hbm.at[s], dst_v)
    plsc.subcore_barrier()

    for b in range(NBUF):
        pltpu.async_copy(y_sh.at[dst_v.at[b]], row_buf.at[b], sem.at[b])

    def outer(j0, carry):
        for b in range(NBUF):
            j = j0 * NBUF + b
            pltpu.make_async_copy(
                y_sh.at[dst_v.at[j]], row_buf.at[b], sem.at[b]).wait()
            pltpu.sync_copy(row_buf.at[b], acc.at[src_v.at[j]], add=True)
            jn = j + NBUF

            @pl.when(jn < CHUNKS_PER_TILE)
            def _():
                pltpu.async_copy(
                    y_sh.at[dst_v.at[jn]], row_buf.at[b], sem.at[b])
        return carry

    lax.fori_loop(0, CHUNKS_PER_TILE // NBUF, outer, 0)
    plsc.subcore_barrier()

    @pl.when(s < NS - 1)
    def _():
        pltpu.sync_copy(acc.at[stripe], out_hbm.at[stripe])

    @pl.when(s == NS - 1)
    def _():
        last = pl.ds((NS - 1) * ROWS_PER_TILE, LAST_ROWS)
        pltpu.sync_copy(acc.at[last], out_hbm.at[last])


def _scatter(y, src_p, dst_p, zeros):
    mesh = plsc.VectorSubcoreMesh(
        core_axis_name="c", subcore_axis_name="s", num_cores=1)
    f = pl.kernel(
        _sc_body,
        out_type=jax.ShapeDtypeStruct((N, OUT_F), jnp.float32),
        mesh=mesh,
        scratch_types=[
            pltpu.VMEM((CHUNKS_PER_TILE, CHUNK), jnp.int32),
            pltpu.VMEM((CHUNKS_PER_TILE, CHUNK), jnp.int32),
            pltpu.VMEM((NBUF, CHUNK, OUT_F), jnp.float32),
            pltpu.VMEM_SHARED((N_PAD, OUT_F), jnp.float32),
            pltpu.VMEM_SHARED((N, OUT_F), jnp.float32),
            pltpu.SemaphoreType.DMA((NBUF,)),
        ],
        compiler_params=pltpu.CompilerParams(use_tc_tiling_on_sc=False),
    )
    return f(y, src_p, dst_p, zeros)


def kernel(x, edge_indices, W, a):
    del a
    x2 = x[0]
    y = _matmul(x2, W)

    pad = E_PAD - E
    src_p = jnp.concatenate(
        [edge_indices[0], jnp.full((pad,), N, jnp.int32)]
    ).reshape(NS, CHUNKS_PER_TILE, CHUNK)
    dst_p = jnp.concatenate(
        [edge_indices[1], jnp.zeros((pad,), jnp.int32)]
    ).reshape(NS, CHUNKS_PER_TILE, CHUNK)
    zeros = jnp.zeros((N_PAD, OUT_F), jnp.float32)

    out = _scatter(y, src_p, dst_p, zeros)
    return out[None]


# speedup vs baseline: 310.0955x; 1.0015x over previous
import functools

import jax
import jax.numpy as jnp
from jax import lax
from jax.experimental import pallas as pl
from jax.experimental.pallas import tpu as pltpu
from jax.experimental.pallas import tpu_sc as plsc

N = 10000
E = 160000
IN_F = 128
HEADS = 8
OUT_F = 16

NS = 16
CHUNK = 128
CHUNKS_PER_TILE = 80
E_PAD = NS * CHUNKS_PER_TILE * CHUNK
N_PAD = 10112
ROWS_PER_TILE = N_PAD // NS
LAST_ROWS = N - (NS - 1) * ROWS_PER_TILE

MM_BLOCK = 1000
NBUF = 8


def _mm_body(x_ref, w_ref, y_ref):
    w = w_ref[...]
    w2 = w[:, 0:OUT_F]
    for h in range(1, HEADS):
        w2 = w2 + w[:, h * OUT_F:(h + 1) * OUT_F]
    y_ref[...] = jnp.dot(x_ref[0], w2,
                         preferred_element_type=jnp.float32) * (1.0 / HEADS)


def _matmul(x, W):
    return pl.pallas_call(
        _mm_body,
        grid=(N // MM_BLOCK,),
        in_specs=[
            pl.BlockSpec((1, MM_BLOCK, IN_F), lambda i: (0, i, 0)),
            pl.BlockSpec((IN_F, IN_F), lambda i: (0, 0)),
        ],
        out_specs=pl.BlockSpec((MM_BLOCK, OUT_F), lambda i: (i, 0)),
        out_shape=jax.ShapeDtypeStruct((N, OUT_F), jnp.float32),
    )(x, W)


def _sc_body(y_hbm, src_hbm, dst_hbm, zeros_hbm, out_hbm,
             src_v, dst_v, row_buf, acc, y_sh, sem):
    s = lax.axis_index("s")
    stripe = pl.ds(s * ROWS_PER_TILE, ROWS_PER_TILE)
    pltpu.sync_copy(zeros_hbm.at[stripe], acc.at[stripe])

    @pl.when(s < NS - 1)
    def _():
        pltpu.sync_copy(y_hbm.at[stripe], y_sh.at[stripe])

    @pl.when(s == NS - 1)
    def _():
        last = pl.ds((NS - 1) * ROWS_PER_TILE, LAST_ROWS)
        pltpu.sync_copy(y_hbm.at[last], y_sh.at[last])

    pltpu.sync_copy(src_hbm.at[s], src_v)
    pltpu.sync_copy(dst_hbm.at[s], dst_v)
    plsc.subcore_barrier()

    for b in range(NBUF):
        pltpu.async_copy(y_sh.at[dst_v.at[b]], row_buf.at[b], sem.at[b])

    def outer(j0, carry):
        for b in range(NBUF):
            j = j0 * NBUF + b
            pltpu.make_async_copy(
                y_sh.at[dst_v.at[j]], row_buf.at[b], sem.at[b]).wait()
            pltpu.sync_copy(row_buf.at[b], acc.at[src_v.at[j]], add=True)
            jn = j + NBUF

            @pl.when(jn < CHUNKS_PER_TILE)
            def _():
                pltpu.async_copy(
                    y_sh.at[dst_v.at[jn]], row_buf.at[b], sem.at[b])
        return carry

    lax.fori_loop(0, CHUNKS_PER_TILE // NBUF, outer, 0)
    plsc.subcore_barrier()

    @pl.when(s < NS - 1)
    def _():
        pltpu.sync_copy(acc.at[stripe], out_hbm.at[0, stripe])

    @pl.when(s == NS - 1)
    def _():
        last = pl.ds((NS - 1) * ROWS_PER_TILE, LAST_ROWS)
        pltpu.sync_copy(acc.at[last], out_hbm.at[0, last])


def _scatter(y, src_p, dst_p, zeros):
    mesh = plsc.VectorSubcoreMesh(
        core_axis_name="c", subcore_axis_name="s", num_cores=1)
    f = pl.kernel(
        _sc_body,
        out_type=jax.ShapeDtypeStruct((1, N, OUT_F), jnp.float32),
        mesh=mesh,
        scratch_types=[
            pltpu.VMEM((CHUNKS_PER_TILE, CHUNK), jnp.int32),
            pltpu.VMEM((CHUNKS_PER_TILE, CHUNK), jnp.int32),
            pltpu.VMEM((NBUF, CHUNK, OUT_F), jnp.float32),
            pltpu.VMEM_SHARED((N_PAD, OUT_F), jnp.float32),
            pltpu.VMEM_SHARED((N, OUT_F), jnp.float32),
            pltpu.SemaphoreType.DMA((NBUF,)),
        ],
        compiler_params=pltpu.CompilerParams(use_tc_tiling_on_sc=False),
    )
    return f(y, src_p, dst_p, zeros)


def kernel(x, edge_indices, W, a):
    del a
    y = _matmul(x, W)

    pad = E_PAD - E
    src_p = jnp.concatenate(
        [edge_indices[0], jnp.full((pad,), N, jnp.int32)]
    ).reshape(NS, CHUNKS_PER_TILE, CHUNK)
    dst_p = jnp.concatenate(
        [edge_indices[1], jnp.zeros((pad,), jnp.int32)]
    ).reshape(NS, CHUNKS_PER_TILE, CHUNK)
    zeros = jnp.zeros((N_PAD, OUT_F), jnp.float32)

    return _scatter(y, src_p, dst_p, zeros)


# speedup vs baseline: 329.6151x; 1.0629x over previous
import functools

import jax
import jax.numpy as jnp
from jax import lax
from jax.experimental import pallas as pl
from jax.experimental.pallas import tpu as pltpu
from jax.experimental.pallas import tpu_sc as plsc

N = 10000
E = 160000
IN_F = 128
HEADS = 8
OUT_F = 16

NS = 16
CHUNK = 128
CHUNKS_PER_TILE = 80
E_PAD = NS * CHUNKS_PER_TILE * CHUNK
N_PAD = 10112
ROWS_PER_TILE = N_PAD // NS
LAST_ROWS = N - (NS - 1) * ROWS_PER_TILE
Y_ROWS_PER_TILE = 78
Y_LAST_ROWS = N // 8 - (NS - 1) * Y_ROWS_PER_TILE

MM_BLOCK = 2000
NBUF = 8


def _mm_body(x_ref, w_ref, y_ref):
    w = w_ref[...]
    w2 = w[:, 0:OUT_F]
    for h in range(1, HEADS):
        w2 = w2 + w[:, h * OUT_F:(h + 1) * OUT_F]
    y = jnp.dot(x_ref[0], w2, preferred_element_type=jnp.float32)
    y_ref[...] = jnp.concatenate(
        [y * (1.0 / HEADS), jnp.zeros((MM_BLOCK, IN_F - OUT_F), jnp.float32)],
        axis=1)


def _matmul(x, W):
    return pl.pallas_call(
        _mm_body,
        grid=(N // MM_BLOCK,),
        in_specs=[
            pl.BlockSpec((1, MM_BLOCK, IN_F), lambda i: (0, i, 0)),
            pl.BlockSpec((IN_F, IN_F), lambda i: (0, 0)),
        ],
        out_specs=pl.BlockSpec((MM_BLOCK, IN_F), lambda i: (i, 0)),
        out_shape=jax.ShapeDtypeStruct((N, IN_F), jnp.float32),
    )(x, W)


def _sc_body(y_hbm, src_hbm, dst_hbm, zeros_hbm, out_hbm,
             src_v, dst_v, row_buf, acc, y_sh, sem):
    s = lax.axis_index("s")
    stripe = pl.ds(s * ROWS_PER_TILE, ROWS_PER_TILE)
    pltpu.sync_copy(zeros_hbm.at[stripe], acc.at[stripe])

    @pl.when(s < NS - 1)
    def _():
        pltpu.sync_copy(y_hbm.at[stripe, pl.ds(0, OUT_F)], y_sh.at[stripe])

    @pl.when(s == NS - 1)
    def _():
        last = pl.ds((NS - 1) * ROWS_PER_TILE, LAST_ROWS)
        pltpu.sync_copy(y_hbm.at[last, pl.ds(0, OUT_F)], y_sh.at[last])

    pltpu.sync_copy(src_hbm.at[s], src_v)
    pltpu.sync_copy(dst_hbm.at[s], dst_v)
    plsc.subcore_barrier()

    for b in range(NBUF):
        pltpu.async_copy(y_sh.at[dst_v.at[b]], row_buf.at[b], sem.at[b])

    def outer(j0, carry):
        for b in range(NBUF):
            j = j0 * NBUF + b
            pltpu.make_async_copy(
                y_sh.at[dst_v.at[j]], row_buf.at[b], sem.at[b]).wait()
            pltpu.sync_copy(row_buf.at[b], acc.at[src_v.at[j]], add=True)
            jn = j + NBUF

            @pl.when(jn < CHUNKS_PER_TILE)
            def _():
                pltpu.async_copy(
                    y_sh.at[dst_v.at[jn]], row_buf.at[b], sem.at[b])
        return carry

    lax.fori_loop(0, CHUNKS_PER_TILE // NBUF, outer, 0)
    plsc.subcore_barrier()

    @pl.when(s < NS - 1)
    def _():
        pltpu.sync_copy(acc.at[stripe], out_hbm.at[0, stripe])

    @pl.when(s == NS - 1)
    def _():
        last = pl.ds((NS - 1) * ROWS_PER_TILE, LAST_ROWS)
        pltpu.sync_copy(acc.at[last], out_hbm.at[0, last])


def _scatter(y, src_p, dst_p, zeros):
    mesh = plsc.VectorSubcoreMesh(
        core_axis_name="c", subcore_axis_name="s", num_cores=1)
    f = pl.kernel(
        _sc_body,
        out_type=jax.ShapeDtypeStruct((1, N, OUT_F), jnp.float32),
        mesh=mesh,
        scratch_types=[
            pltpu.VMEM((CHUNKS_PER_TILE, CHUNK), jnp.int32),
            pltpu.VMEM((CHUNKS_PER_TILE, CHUNK), jnp.int32),
            pltpu.VMEM((NBUF, CHUNK, OUT_F), jnp.float32),
            pltpu.VMEM_SHARED((N_PAD, OUT_F), jnp.float32),
            pltpu.VMEM_SHARED((N, OUT_F), jnp.float32),
            pltpu.SemaphoreType.DMA((NBUF,)),
        ],
        compiler_params=pltpu.CompilerParams(use_tc_tiling_on_sc=False),
    )
    return f(y, src_p, dst_p, zeros)


def kernel(x, edge_indices, W, a):
    del a
    y = _matmul(x, W)

    pad = E_PAD - E
    src_p = jnp.concatenate(
        [edge_indices[0], jnp.full((pad,), N, jnp.int32)]
    ).reshape(NS, CHUNKS_PER_TILE, CHUNK)
    dst_p = jnp.concatenate(
        [edge_indices[1], jnp.zeros((pad,), jnp.int32)]
    ).reshape(NS, CHUNKS_PER_TILE, CHUNK)
    zeros = jnp.zeros((N_PAD, OUT_F), jnp.float32)

    return _scatter(y, src_p, dst_p, zeros)


# speedup vs baseline: 334.9089x; 1.0161x over previous
import functools

import jax
import jax.numpy as jnp
from jax import lax
from jax.experimental import pallas as pl
from jax.experimental.pallas import tpu as pltpu
from jax.experimental.pallas import tpu_sc as plsc

N = 10000
E = 160000
IN_F = 128
HEADS = 8
OUT_F = 16

NS = 16
CHUNK = 128
CHUNKS_PER_TILE = 80
E_PAD = NS * CHUNKS_PER_TILE * CHUNK
N_PAD = 10112
ROWS_PER_TILE = N_PAD // NS
LAST_ROWS = N - (NS - 1) * ROWS_PER_TILE
Y_ROWS_PER_TILE = 78
Y_LAST_ROWS = N // 8 - (NS - 1) * Y_ROWS_PER_TILE

MM_BLOCK = 2000
NBUF = 8


def _mm_body(x_ref, w_ref, y_ref):
    w = w_ref[...]
    w2 = w[:, 0:OUT_F]
    for h in range(1, HEADS):
        w2 = w2 + w[:, h * OUT_F:(h + 1) * OUT_F]
    y = jnp.dot(x_ref[0], w2, preferred_element_type=jnp.float32)
    y_ref[...] = jnp.concatenate(
        [y * (1.0 / HEADS), jnp.zeros((MM_BLOCK, IN_F - OUT_F), jnp.float32)],
        axis=1)


def _matmul(x, W):
    return pl.pallas_call(
        _mm_body,
        grid=(N // MM_BLOCK,),
        in_specs=[
            pl.BlockSpec((1, MM_BLOCK, IN_F), lambda i: (0, i, 0)),
            pl.BlockSpec((IN_F, IN_F), lambda i: (0, 0)),
        ],
        out_specs=pl.BlockSpec((MM_BLOCK, IN_F), lambda i: (i, 0)),
        out_shape=jax.ShapeDtypeStruct((N, IN_F), jnp.float32),
    )(x, W)


def _sc_body(y_hbm, src_hbm, dst_hbm, out_hbm,
             src_v, dst_v, row_buf, acc, y_sh, sem):
    s = lax.axis_index("s")
    stripe = pl.ds(s * ROWS_PER_TILE, ROWS_PER_TILE)
    def zero_row(i, carry):
        row_buf[0, i] = jnp.zeros((OUT_F,), jnp.float32)
        return carry

    lax.fori_loop(0, CHUNK, zero_row, 0)
    for q in range(4):
        pltpu.sync_copy(
            row_buf.at[0],
            acc.at[pl.ds(s * ROWS_PER_TILE + q * CHUNK, CHUNK)])
    pltpu.sync_copy(
        row_buf.at[0, pl.ds(0, ROWS_PER_TILE - 4 * CHUNK)],
        acc.at[pl.ds(s * ROWS_PER_TILE + 4 * CHUNK, ROWS_PER_TILE - 4 * CHUNK)])

    @pl.when(s < NS - 1)
    def _():
        pltpu.sync_copy(y_hbm.at[stripe, pl.ds(0, OUT_F)], y_sh.at[stripe])

    @pl.when(s == NS - 1)
    def _():
        last = pl.ds((NS - 1) * ROWS_PER_TILE, LAST_ROWS)
        pltpu.sync_copy(y_hbm.at[last, pl.ds(0, OUT_F)], y_sh.at[last])

    pltpu.sync_copy(src_hbm.at[s], src_v)
    pltpu.sync_copy(dst_hbm.at[s], dst_v)
    plsc.subcore_barrier()

    for b in range(NBUF):
        pltpu.async_copy(y_sh.at[dst_v.at[b]], row_buf.at[b], sem.at[b])

    def outer(j0, carry):
        for b in range(NBUF):
            j = j0 * NBUF + b
            pltpu.make_async_copy(
                y_sh.at[dst_v.at[j]], row_buf.at[b], sem.at[b]).wait()
            pltpu.sync_copy(row_buf.at[b], acc.at[src_v.at[j]], add=True)
            jn = j + NBUF

            @pl.when(jn < CHUNKS_PER_TILE)
            def _():
                pltpu.async_copy(
                    y_sh.at[dst_v.at[jn]], row_buf.at[b], sem.at[b])
        return carry

    lax.fori_loop(0, CHUNKS_PER_TILE // NBUF, outer, 0)
    plsc.subcore_barrier()

    @pl.when(s < NS - 1)
    def _():
        pltpu.sync_copy(acc.at[stripe], out_hbm.at[0, stripe, pl.ds(0, OUT_F)])

    @pl.when(s == NS - 1)
    def _():
        last = pl.ds((NS - 1) * ROWS_PER_TILE, LAST_ROWS)
        pltpu.sync_copy(acc.at[last], out_hbm.at[0, last, pl.ds(0, OUT_F)])


def _scatter(y, src_p, dst_p):
    mesh = plsc.VectorSubcoreMesh(
        core_axis_name="c", subcore_axis_name="s", num_cores=1)
    f = pl.kernel(
        _sc_body,
        out_type=jax.ShapeDtypeStruct((1, N, IN_F), jnp.float32),
        mesh=mesh,
        scratch_types=[
            pltpu.VMEM((CHUNKS_PER_TILE, CHUNK), jnp.int32),
            pltpu.VMEM((CHUNKS_PER_TILE, CHUNK), jnp.int32),
            pltpu.VMEM((NBUF, CHUNK, OUT_F), jnp.float32),
            pltpu.VMEM_SHARED((N_PAD, OUT_F), jnp.float32),
            pltpu.VMEM_SHARED((N, OUT_F), jnp.float32),
            pltpu.SemaphoreType.DMA((NBUF,)),
        ],
        compiler_params=pltpu.CompilerParams(use_tc_tiling_on_sc=False),
    )
    return f(y, src_p, dst_p)


def kernel(x, edge_indices, W, a):
    del a
    y = _matmul(x, W)

    pad = E_PAD - E
    src_p = jnp.concatenate(
        [edge_indices[0], jnp.full((pad,), N, jnp.int32)]
    ).reshape(NS, CHUNKS_PER_TILE, CHUNK)
    dst_p = jnp.concatenate(
        [edge_indices[1], jnp.zeros((pad,), jnp.int32)]
    ).reshape(NS, CHUNKS_PER_TILE, CHUNK)
    out_wide = _scatter(y, src_p, dst_p)
    return out_wide[:, :, :OUT_F]


# speedup vs baseline: 350.9037x; 1.0478x over previous
import functools

import jax
import jax.numpy as jnp
from jax import lax
from jax.experimental import pallas as pl
from jax.experimental.pallas import tpu as pltpu
from jax.experimental.pallas import tpu_sc as plsc

N = 10000
E = 160000
IN_F = 128
HEADS = 8
OUT_F = 16

NS = 16
CHUNK = 128
E_ROWS = E // CHUNK
BASE_ROWS = E_ROWS // NS
EXTRA_TILES = E_ROWS - BASE_ROWS * NS
MAX_ROWS = BASE_ROWS + 1
N_PAD = 10112
ROWS_PER_TILE = N_PAD // NS
LAST_ROWS = N - (NS - 1) * ROWS_PER_TILE
Y_ROWS_PER_TILE = 78
Y_LAST_ROWS = N // 8 - (NS - 1) * Y_ROWS_PER_TILE

MM_BLOCK = 2000
NBUF = 8


def _mm_body(x_ref, w_ref, y_ref):
    w = w_ref[...]
    w2 = w[:, 0:OUT_F]
    for h in range(1, HEADS):
        w2 = w2 + w[:, h * OUT_F:(h + 1) * OUT_F]
    y = jnp.dot(x_ref[0], w2, preferred_element_type=jnp.float32)
    y_ref[...] = jnp.concatenate(
        [y * (1.0 / HEADS), jnp.zeros((MM_BLOCK, IN_F - OUT_F), jnp.float32)],
        axis=1)


def _matmul(x, W):
    return pl.pallas_call(
        _mm_body,
        grid=(N // MM_BLOCK,),
        in_specs=[
            pl.BlockSpec((1, MM_BLOCK, IN_F), lambda i: (0, i, 0)),
            pl.BlockSpec((IN_F, IN_F), lambda i: (0, 0)),
        ],
        out_specs=pl.BlockSpec((MM_BLOCK, IN_F), lambda i: (i, 0)),
        out_shape=jax.ShapeDtypeStruct((N, IN_F), jnp.float32),
    )(x, W)


def _sc_body(y_hbm, src_hbm, dst_hbm, out_hbm,
             src_v, dst_v, row_buf, acc, y_sh, sem):
    s = lax.axis_index("s")
    nfirst = NS - EXTRA_TILES
    tn = BASE_ROWS + jnp.where(s < nfirst, 0, 1)
    row0 = s * BASE_ROWS + jnp.maximum(s - nfirst, 0)
    stripe = pl.ds(s * ROWS_PER_TILE, ROWS_PER_TILE)
    def zero_row(i, carry):
        row_buf[0, i] = jnp.zeros((OUT_F,), jnp.float32)
        return carry

    lax.fori_loop(0, CHUNK, zero_row, 0)
    for q in range(4):
        pltpu.sync_copy(
            row_buf.at[0],
            acc.at[pl.ds(s * ROWS_PER_TILE + q * CHUNK, CHUNK)])
    pltpu.sync_copy(
        row_buf.at[0, pl.ds(0, ROWS_PER_TILE - 4 * CHUNK)],
        acc.at[pl.ds(s * ROWS_PER_TILE + 4 * CHUNK, ROWS_PER_TILE - 4 * CHUNK)])

    @pl.when(s < NS - 1)
    def _():
        pltpu.sync_copy(y_hbm.at[stripe, pl.ds(0, OUT_F)], y_sh.at[stripe])

    @pl.when(s == NS - 1)
    def _():
        last = pl.ds((NS - 1) * ROWS_PER_TILE, LAST_ROWS)
        pltpu.sync_copy(y_hbm.at[last, pl.ds(0, OUT_F)], y_sh.at[last])

    @pl.when(s < nfirst)
    def _():
        pltpu.sync_copy(src_hbm.at[pl.ds(row0, BASE_ROWS)],
                        src_v.at[pl.ds(0, BASE_ROWS)])
        pltpu.sync_copy(dst_hbm.at[pl.ds(row0, BASE_ROWS)],
                        dst_v.at[pl.ds(0, BASE_ROWS)])

    @pl.when(s >= nfirst)
    def _():
        pltpu.sync_copy(src_hbm.at[pl.ds(row0, MAX_ROWS)], src_v)
        pltpu.sync_copy(dst_hbm.at[pl.ds(row0, MAX_ROWS)], dst_v)

    plsc.subcore_barrier()

    for b in range(NBUF):
        @pl.when(b < tn)
        def _():
            pltpu.async_copy(y_sh.at[dst_v.at[b]], row_buf.at[b], sem.at[b])

    def outer(j0, carry):
        for b in range(NBUF):
            j = j0 * NBUF + b

            @pl.when(j < tn)
            def _():
                pltpu.make_async_copy(
                    y_sh.at[dst_v.at[j]], row_buf.at[b], sem.at[b]).wait()
                pltpu.sync_copy(row_buf.at[b], acc.at[src_v.at[j]], add=True)
                jn = j + NBUF

                @pl.when(jn < tn)
                def _():
                    pltpu.async_copy(
                        y_sh.at[dst_v.at[jn]], row_buf.at[b], sem.at[b])
        return carry

    lax.fori_loop(0, (MAX_ROWS + NBUF - 1) // NBUF, outer, 0)
    plsc.subcore_barrier()

    @pl.when(s < NS - 1)
    def _():
        pltpu.sync_copy(acc.at[stripe], out_hbm.at[0, stripe, pl.ds(0, OUT_F)])

    @pl.when(s == NS - 1)
    def _():
        last = pl.ds((NS - 1) * ROWS_PER_TILE, LAST_ROWS)
        pltpu.sync_copy(acc.at[last], out_hbm.at[0, last, pl.ds(0, OUT_F)])


def _scatter(y, src_p, dst_p):
    mesh = plsc.VectorSubcoreMesh(
        core_axis_name="c", subcore_axis_name="s", num_cores=1)
    f = pl.kernel(
        _sc_body,
        out_type=jax.ShapeDtypeStruct((1, N, IN_F), jnp.float32),
        mesh=mesh,
        scratch_types=[
            pltpu.VMEM((MAX_ROWS, CHUNK), jnp.int32),
            pltpu.VMEM((MAX_ROWS, CHUNK), jnp.int32),
            pltpu.VMEM((NBUF, CHUNK, OUT_F), jnp.float32),
            pltpu.VMEM_SHARED((N_PAD, OUT_F), jnp.float32),
            pltpu.VMEM_SHARED((N, OUT_F), jnp.float32),
            pltpu.SemaphoreType.DMA((NBUF,)),
        ],
        compiler_params=pltpu.CompilerParams(use_tc_tiling_on_sc=False),
    )
    return f(y, src_p, dst_p)


def kernel(x, edge_indices, W, a):
    del a
    y = _matmul(x, W)

    src_p = edge_indices[0].reshape(E_ROWS, CHUNK)
    dst_p = edge_indices[1].reshape(E_ROWS, CHUNK)
    out_wide = _scatter(y, src_p, dst_p)
    return out_wide[:, :, :OUT_F]
